# Initial kernel scaffold; baseline (speedup 1.0000x reference)
import jax
import jax.numpy as jnp
from jax.experimental import pallas as pl


def kernel(node_features, edge_index, edge_type, node_to_graph_id, edge_W, W_ih, W_hh, b_ih, b_hh):
    raise NotImplementedError("write your pallas kernel here")


# speedup vs baseline: 16.4972x; 16.4972x over previous
import functools

import jax
import jax.numpy as jnp
from jax import lax
from jax.experimental import pallas as pl
from jax.experimental.pallas import tpu as pltpu
from jax.experimental.pallas import tpu_sc as plsc

NC = 2
NS = 16
NW = NC * NS

BATCH = 128


def _sc_aggregate(n_nodes, d, n_tables, e_pad):
    per_tile = e_pad // NW
    chunks = per_tile // BATCH
    zrows_per_tile = -(-(n_nodes + 1) // (NS * BATCH)) * BATCH
    sp_rows = NS * zrows_per_tile
    zchunks = zrows_per_tile // BATCH
    assert sp_rows * d * 4 <= 6 * 1024 * 1024

    mesh = plsc.VectorSubcoreMesh(core_axis_name="c", subcore_axis_name="s")

    @functools.partial(
        pl.kernel,
        mesh=mesh,
        compiler_params=pltpu.CompilerParams(use_tc_tiling_on_sc=False),
        out_type=jax.ShapeDtypeStruct((NC, sp_rows, d), jnp.float32),
        scratch_types=[
            pltpu.VMEM((chunks, BATCH), jnp.int32),
            pltpu.VMEM((chunks, BATCH), jnp.int32),
            pltpu.VMEM((BATCH, d), jnp.float32),
            pltpu.VMEM_SHARED((sp_rows, d), jnp.float32),
            pltpu.SemaphoreType.DMA,
        ],
    )
    def sc_kernel(tab, ksrc, kdst, part, src_v, dst_v, rows_v, acc_sp, sem):
        c = lax.axis_index("c")
        s = lax.axis_index("s")
        w = c * NS + s

        z16 = jnp.zeros((16,), jnp.float32)

        def _zero_row(i, carry):
            for j in range(d // 16):
                rows_v[i, pl.ds(j * 16, 16)] = z16
            return carry

        lax.fori_loop(0, BATCH, _zero_row, 0)

        def _zero_sp(k, carry):
            pltpu.sync_copy(
                rows_v, acc_sp.at[pl.ds(s * zrows_per_tile + k * BATCH, BATCH)]
            )
            return carry

        lax.fori_loop(0, zchunks, _zero_sp, 0)
        plsc.subcore_barrier()

        pltpu.sync_copy(ksrc.at[w], src_v)
        pltpu.sync_copy(kdst.at[w], dst_v)

        def _edge_batch(j, carry):
            pltpu.async_copy(tab.at[src_v.at[j]], rows_v, sem).wait()
            pltpu.sync_copy(rows_v, acc_sp.at[dst_v.at[j]], add=True)
            return carry

        lax.fori_loop(0, chunks, _edge_batch, 0)
        plsc.subcore_barrier()

        def _out(k, carry):
            r0 = s * zrows_per_tile + k * BATCH
            pltpu.sync_copy(acc_sp.at[pl.ds(r0, BATCH)], rows_v)
            pltpu.sync_copy(rows_v, part.at[c, pl.ds(r0, BATCH)])
            return carry

        lax.fori_loop(0, zchunks, _out, 0)

    return sc_kernel, sp_rows


def _tc_round(n_nodes, d, n_types, row_block, emit_tables):
    grid = (n_nodes // row_block,)

    def body(h_ref, p0, p1, ew, wih, whh, bih, bhh, out_ref, *tab_ref):
        h = h_ref[...]
        f32 = jnp.float32
        m = p0[...] + p1[...]
        gi = jnp.dot(m, wih[...], preferred_element_type=f32) + bih[...]
        gh = jnp.dot(h, whh[...], preferred_element_type=f32) + bhh[...]
        r = jax.nn.sigmoid(gi[:, :d] + gh[:, :d])
        z = jax.nn.sigmoid(gi[:, d:2 * d] + gh[:, d:2 * d])
        n = jnp.tanh(gi[:, 2 * d:] + r * gh[:, 2 * d:])
        h_new = (1.0 - z) * n + z * h
        out_ref[...] = h_new
        if emit_tables:
            for t in range(n_types):
                tab_ref[0][t] = jnp.dot(h_new, ew[t],
                                        preferred_element_type=f32)

    row_spec = pl.BlockSpec((row_block, d), lambda i: (i, 0))
    out_shapes = [jax.ShapeDtypeStruct((n_nodes, d), jnp.float32)]
    out_specs = [row_spec]
    if emit_tables:
        out_shapes.append(
            jax.ShapeDtypeStruct((n_types, n_nodes, d), jnp.float32))
        out_specs.append(
            pl.BlockSpec((n_types, row_block, d), lambda i: (0, i, 0)))
    return pl.pallas_call(
        body,
        grid=grid,
        in_specs=[
            row_spec, row_spec, row_spec,
            pl.BlockSpec((n_types, d, d), lambda i: (0, 0, 0)),
            pl.BlockSpec((d, 3 * d), lambda i: (0, 0)),
            pl.BlockSpec((d, 3 * d), lambda i: (0, 0)),
            pl.BlockSpec((1, 3 * d), lambda i: (0, 0)),
            pl.BlockSpec((1, 3 * d), lambda i: (0, 0)),
        ],
        out_specs=out_specs,
        out_shape=out_shapes,
    )


def _tc_tables(n_nodes, d, n_types, row_block):
    grid = (n_nodes // row_block,)

    def body(h_ref, ew, tab_ref):
        h = h_ref[...]
        for t in range(n_types):
            tab_ref[t] = jnp.dot(h, ew[t], preferred_element_type=jnp.float32)

    return pl.pallas_call(
        body,
        grid=grid,
        in_specs=[
            pl.BlockSpec((row_block, d), lambda i: (i, 0)),
            pl.BlockSpec((n_types, d, d), lambda i: (0, 0, 0)),
        ],
        out_specs=pl.BlockSpec((n_types, row_block, d), lambda i: (0, i, 0)),
        out_shape=jax.ShapeDtypeStruct((n_types, n_nodes, d), jnp.float32),
    )


def kernel(node_features, edge_index, edge_type, node_to_graph_id,
           edge_W, W_ih, W_hh, b_ih, b_hh):
    n_nodes, d = node_features.shape
    e = edge_index.shape[1]
    n_types = edge_W.shape[0]
    num_steps = 4

    per_tile = -(-e // NW)
    chunks = -(-per_tile // BATCH)
    e_pad = NW * chunks * BATCH
    pad = e_pad - e

    src = edge_index[0]
    dst = edge_index[1]
    ksrc = edge_type * n_nodes + src
    ksrc = jnp.pad(ksrc, (0, pad)).reshape(NW, chunks, BATCH)
    kdst = jnp.pad(dst, (0, pad), constant_values=n_nodes)
    kdst = kdst.reshape(NW, chunks, BATCH)

    sc_step, sp_rows = _sc_aggregate(n_nodes, d, n_types, e_pad)
    tc_mid = _tc_round(n_nodes, d, n_types, 400, emit_tables=True)
    tc_last = _tc_round(n_nodes, d, n_types, 400, emit_tables=False)
    tc_prep = _tc_tables(n_nodes, d, n_types, 400)

    bih = b_ih.reshape(1, 3 * d)
    bhh = b_hh.reshape(1, 3 * d)

    h = node_features
    tables = tc_prep(h, edge_W)
    for step in range(num_steps):
        part = sc_step(tables.reshape(n_types * n_nodes, d), ksrc, kdst)
        if step < num_steps - 1:
            h, tables = tc_mid(h, part[0], part[1], edge_W,
                               W_ih, W_hh, bih, bhh)
        else:
            (h,) = tc_last(h, part[0], part[1], edge_W, W_ih, W_hh, bih, bhh)
    return h
